# Initial kernel scaffold; baseline (speedup 1.0000x reference)
import jax
import jax.numpy as jnp
from jax.experimental import pallas as pl


def kernel(query_embed, kb, Wq, bq, W1, b1, g1, beta1, W2, b2, g2, beta2):
    raise NotImplementedError("write your pallas kernel here")


# speedup vs baseline: 1.7176x; 1.7176x over previous
import functools

import jax
import jax.numpy as jnp
from jax import lax
from jax.experimental import pallas as pl
from jax.experimental.pallas import tpu as pltpu
from jax.experimental.pallas import tpu_sc as plsc

TK = 8
_KBLK = 2048


def _qproj_body(x_ref, w_ref, b_ref, o_ref):
    q = jnp.dot(x_ref[...], w_ref[...], preferred_element_type=jnp.float32)
    q = q + b_ref[...]
    n = jnp.sqrt(jnp.sum(q * q, axis=1, keepdims=True))
    o_ref[...] = q / n


def _qproj(x, w, b2d):
    B, D = x.shape
    return pl.pallas_call(
        _qproj_body,
        out_shape=jax.ShapeDtypeStruct((B, D), jnp.float32),
    )(x, w, b2d)


def _sim_topk_body(qn_ref, kb_ref, vals_ref, idx_ref, *, B, K, kblk):
    k = pl.program_id(0)

    @pl.when(k == 0)
    def _init():
        vals_ref[...] = jnp.full((B, TK), -jnp.inf, jnp.float32)
        idx_ref[...] = jnp.zeros((B, TK), jnp.int32)

    kbb = kb_ref[...]
    n2 = jnp.sum(kbb * kbb, axis=1, keepdims=True)
    kbn = kbb / jnp.sqrt(n2)
    s = lax.dot_general(qn_ref[...], kbn, (((1,), (1,)), ((), ())),
                        preferred_element_type=jnp.float32)

    lane = lax.broadcasted_iota(jnp.int32, (1, kblk), 1)
    col_ok = (k * kblk + lane) < K
    x = jnp.where(col_ok, s, -jnp.inf)

    bv, bi = [], []
    for _ in range(TK):
        m = jnp.max(x, axis=1, keepdims=True)
        am = jnp.min(jnp.where(x == m, lane, kblk), axis=1, keepdims=True)
        bv.append(m)
        bi.append(am + k * kblk)
        x = jnp.where(lane == am, -jnp.inf, x)

    cv = jnp.concatenate([vals_ref[...]] + bv, axis=1)
    ci = jnp.concatenate([idx_ref[...]] + bi, axis=1)
    lane16 = lax.broadcasted_iota(jnp.int32, (1, 2 * TK), 1)
    nv, ni = [], []
    for _ in range(TK):
        m = jnp.max(cv, axis=1, keepdims=True)
        am = jnp.min(jnp.where(cv == m, lane16, 2 * TK), axis=1, keepdims=True)
        pick = lane16 == am
        nv.append(m)
        ni.append(jnp.max(jnp.where(pick, ci, -1), axis=1, keepdims=True))
        cv = jnp.where(pick, -jnp.inf, cv)
    vals_ref[...] = jnp.concatenate(nv, axis=1)
    idx_ref[...] = jnp.concatenate(ni, axis=1)


def _sim_topk(qn, kb):
    B, D = qn.shape
    K = kb.shape[0]
    nk = pl.cdiv(K, _KBLK)
    body = functools.partial(_sim_topk_body, B=B, K=K, kblk=_KBLK)
    return pl.pallas_call(
        body,
        grid=(nk,),
        in_specs=[
            pl.BlockSpec((B, D), lambda k: (0, 0)),
            pl.BlockSpec((_KBLK, D), lambda k: (k, 0)),
        ],
        out_specs=[
            pl.BlockSpec((B, TK), lambda k: (0, 0)),
            pl.BlockSpec((B, TK), lambda k: (0, 0)),
        ],
        out_shape=[
            jax.ShapeDtypeStruct((B, TK), jnp.float32),
            jax.ShapeDtypeStruct((B, TK), jnp.int32),
        ],
    )(qn, kb)


def _sc_gather(table, idx_flat):
    V, D = table.shape
    N = idx_flat.shape[0]
    info = plsc.get_sparse_core_info()
    nw = info.num_cores * info.num_subcores
    b_per_w = N // nw
    chunk = 128
    mesh = plsc.VectorSubcoreMesh(core_axis_name="c", subcore_axis_name="s")

    @functools.partial(
        pl.kernel,
        mesh=mesh,
        out_type=jax.ShapeDtypeStruct((N, D), jnp.float32),
        scratch_types=[
            pltpu.VMEM((chunk,), jnp.int32),
            pltpu.VMEM((chunk, D), jnp.float32),
            pltpu.SemaphoreType.DMA,
        ],
    )
    def k(table_hbm, idx_hbm, out_hbm, idx_v, rows_v, sem):
        wid = lax.axis_index("s") * info.num_cores + lax.axis_index("c")
        base = wid * b_per_w
        for c in range(b_per_w // chunk):
            off = base + c * chunk
            pltpu.sync_copy(idx_hbm.at[pl.ds(off, chunk)], idx_v)
            pltpu.async_copy(table_hbm.at[idx_v], rows_v, sem).wait()
            pltpu.sync_copy(rows_v, out_hbm.at[pl.ds(off, chunk)])

    return k(table, idx_flat)


def _mlp_body(x_ref, w_ref, b_ref, g_ref, beta_ref, o_ref, acc, *, gelu):
    kk = pl.program_id(1)
    nk = pl.num_programs(1)

    @pl.when(kk == 0)
    def _init():
        acc[...] = jnp.zeros_like(acc)

    acc[...] += jnp.dot(x_ref[...], w_ref[...],
                        preferred_element_type=jnp.float32)

    @pl.when(kk == nk - 1)
    def _fin():
        h = acc[...] + b_ref[...]
        mu = jnp.mean(h, axis=1, keepdims=True)
        var = jnp.mean((h - mu) ** 2, axis=1, keepdims=True)
        h = (h - mu) / jnp.sqrt(var + 1e-5) * g_ref[...] + beta_ref[...]
        if gelu:
            h = 0.5 * h * (1.0 + lax.erf(h / jnp.sqrt(2.0).astype(jnp.float32)))
        o_ref[...] = h


def _mlp(x, w, b2d, g2d, beta2d, gelu):
    B, Kin = x.shape
    H = w.shape[1]
    bblk, kblk = min(256, B), min(512, Kin)
    grid = (B // bblk, Kin // kblk)
    body = functools.partial(_mlp_body, gelu=gelu)
    return pl.pallas_call(
        body,
        grid=grid,
        in_specs=[
            pl.BlockSpec((bblk, kblk), lambda b, k: (b, k)),
            pl.BlockSpec((kblk, H), lambda b, k: (k, 0)),
            pl.BlockSpec((1, H), lambda b, k: (0, 0)),
            pl.BlockSpec((1, H), lambda b, k: (0, 0)),
            pl.BlockSpec((1, H), lambda b, k: (0, 0)),
        ],
        out_specs=pl.BlockSpec((bblk, H), lambda b, k: (b, 0)),
        out_shape=jax.ShapeDtypeStruct((B, H), jnp.float32),
        scratch_shapes=[pltpu.VMEM((bblk, H), jnp.float32)],
    )(x, w, b2d, g2d, beta2d)


def kernel(query_embed, kb, Wq, bq, W1, b1, g1, beta1, W2, b2, g2, beta2):
    B, D = query_embed.shape
    H = W1.shape[1]
    qn = _qproj(query_embed, Wq, bq.reshape(1, D))
    vals, idx = _sim_topk(qn, kb)
    rows = _sc_gather(kb, idx.reshape(B * TK))
    flat = rows.reshape(B, TK * D)
    h1 = _mlp(flat, W1, b1.reshape(1, H), g1.reshape(1, H),
              beta1.reshape(1, H), gelu=True)
    ctx = _mlp(h1, W2, b2.reshape(1, H), g2.reshape(1, H),
               beta2.reshape(1, H), gelu=False)
    return (ctx, idx, vals)


# speedup vs baseline: 2.0584x; 1.1984x over previous
import functools

import jax
import jax.numpy as jnp
from jax import lax
from jax.experimental import pallas as pl
from jax.experimental.pallas import tpu as pltpu
from jax.experimental.pallas import tpu_sc as plsc

TK = 8
_KBLK = 2048


def _qproj_body(x_ref, w_ref, b_ref, o_ref):
    q = jnp.dot(x_ref[...], w_ref[...], preferred_element_type=jnp.float32)
    q = q + b_ref[...]
    n = jnp.sqrt(jnp.sum(q * q, axis=1, keepdims=True))
    o_ref[...] = q / n


def _qproj(x, w, b2d):
    B, D = x.shape
    return pl.pallas_call(
        _qproj_body,
        out_shape=jax.ShapeDtypeStruct((B, D), jnp.float32),
    )(x, w, b2d)


def _sim_topk_body(qn_ref, kb_ref, vals_ref, idx_ref, xs, cand_v, cand_i,
                   *, B, K, kblk):
    k = pl.program_id(0)

    @pl.when(k == 0)
    def _init():
        vals_ref[...] = jnp.full((B, TK), -jnp.inf, jnp.float32)
        idx_ref[...] = jnp.zeros((B, TK), jnp.int32)

    kbb = kb_ref[...]
    n2 = jnp.sum(kbb * kbb, axis=1, keepdims=True)
    kbn = kbb / jnp.sqrt(n2)
    s = lax.dot_general(qn_ref[...], kbn, (((1,), (1,)), ((), ())),
                        preferred_element_type=jnp.float32)

    lane = lax.broadcasted_iota(jnp.int32, (1, kblk), 1)
    col_ok = (k * kblk + lane) < K
    xs[...] = jnp.where(col_ok, s, -jnp.inf)

    rmin = vals_ref[:, TK - 1:TK]
    cnt = jnp.sum((xs[...] > rmin).astype(jnp.int32), axis=1)
    M = jnp.max(cnt)

    cand_v[...] = jnp.full((B, TK), -jnp.inf, jnp.float32)
    cand_i[...] = jnp.zeros((B, TK), jnp.int32)
    lane8 = lax.broadcasted_iota(jnp.int32, (1, TK), 1)
    for t in range(TK):
        @pl.when(t < M)
        def _round():
            x = xs[...]
            m = jnp.max(x, axis=1, keepdims=True)
            am = jnp.min(jnp.where(x == m, lane, kblk), axis=1, keepdims=True)
            cand_v[...] = jnp.where(lane8 == t, m, cand_v[...])
            cand_i[...] = jnp.where(lane8 == t, am + k * kblk, cand_i[...])
            xs[...] = jnp.where(lane == am, -jnp.inf, x)

    @pl.when(M > 0)
    def _merge():
        cv = jnp.concatenate([vals_ref[...], cand_v[...]], axis=1)
        ci = jnp.concatenate([idx_ref[...], cand_i[...]], axis=1)
        lane16 = lax.broadcasted_iota(jnp.int32, (1, 2 * TK), 1)
        nv, ni = [], []
        for _ in range(TK):
            m = jnp.max(cv, axis=1, keepdims=True)
            am = jnp.min(jnp.where(cv == m, lane16, 2 * TK), axis=1,
                         keepdims=True)
            pick = lane16 == am
            nv.append(m)
            ni.append(jnp.max(jnp.where(pick, ci, -1), axis=1, keepdims=True))
            cv = jnp.where(pick, -jnp.inf, cv)
        vals_ref[...] = jnp.concatenate(nv, axis=1)
        idx_ref[...] = jnp.concatenate(ni, axis=1)


def _sim_topk(qn, kb):
    B, D = qn.shape
    K = kb.shape[0]
    nk = pl.cdiv(K, _KBLK)
    body = functools.partial(_sim_topk_body, B=B, K=K, kblk=_KBLK)
    return pl.pallas_call(
        body,
        grid=(nk,),
        in_specs=[
            pl.BlockSpec((B, D), lambda k: (0, 0)),
            pl.BlockSpec((_KBLK, D), lambda k: (k, 0)),
        ],
        out_specs=[
            pl.BlockSpec((B, TK), lambda k: (0, 0)),
            pl.BlockSpec((B, TK), lambda k: (0, 0)),
        ],
        out_shape=[
            jax.ShapeDtypeStruct((B, TK), jnp.float32),
            jax.ShapeDtypeStruct((B, TK), jnp.int32),
        ],
        scratch_shapes=[
            pltpu.VMEM((B, _KBLK), jnp.float32),
            pltpu.VMEM((B, TK), jnp.float32),
            pltpu.VMEM((B, TK), jnp.int32),
        ],
    )(qn, kb)


def _sc_gather(table, idx_flat):
    V, D = table.shape
    N = idx_flat.shape[0]
    info = plsc.get_sparse_core_info()
    nw = info.num_cores * info.num_subcores
    b_per_w = N // nw
    chunk = 128
    mesh = plsc.VectorSubcoreMesh(core_axis_name="c", subcore_axis_name="s")

    @functools.partial(
        pl.kernel,
        mesh=mesh,
        out_type=jax.ShapeDtypeStruct((N, D), jnp.float32),
        scratch_types=[
            pltpu.VMEM((chunk,), jnp.int32),
            pltpu.VMEM((chunk, D), jnp.float32),
            pltpu.SemaphoreType.DMA,
        ],
    )
    def k(table_hbm, idx_hbm, out_hbm, idx_v, rows_v, sem):
        wid = lax.axis_index("s") * info.num_cores + lax.axis_index("c")
        base = wid * b_per_w
        for c in range(b_per_w // chunk):
            off = base + c * chunk
            pltpu.sync_copy(idx_hbm.at[pl.ds(off, chunk)], idx_v)
            pltpu.async_copy(table_hbm.at[idx_v], rows_v, sem).wait()
            pltpu.sync_copy(rows_v, out_hbm.at[pl.ds(off, chunk)])

    return k(table, idx_flat)


def _mlp_body(x_ref, w_ref, b_ref, g_ref, beta_ref, o_ref, acc, *, gelu):
    kk = pl.program_id(1)
    nk = pl.num_programs(1)

    @pl.when(kk == 0)
    def _init():
        acc[...] = jnp.zeros_like(acc)

    acc[...] += jnp.dot(x_ref[...].astype(jnp.bfloat16),
                        w_ref[...].astype(jnp.bfloat16),
                        preferred_element_type=jnp.float32)

    @pl.when(kk == nk - 1)
    def _fin():
        h = acc[...] + b_ref[...]
        mu = jnp.mean(h, axis=1, keepdims=True)
        var = jnp.mean((h - mu) ** 2, axis=1, keepdims=True)
        h = (h - mu) / jnp.sqrt(var + 1e-5) * g_ref[...] + beta_ref[...]
        if gelu:
            h = 0.5 * h * (1.0 + lax.erf(h / jnp.sqrt(2.0).astype(jnp.float32)))
        o_ref[...] = h


def _mlp(x, w, b2d, g2d, beta2d, gelu):
    B, Kin = x.shape
    H = w.shape[1]
    bblk, kblk = min(256, B), min(512, Kin)
    grid = (B // bblk, Kin // kblk)
    body = functools.partial(_mlp_body, gelu=gelu)
    return pl.pallas_call(
        body,
        grid=grid,
        in_specs=[
            pl.BlockSpec((bblk, kblk), lambda b, k: (b, k)),
            pl.BlockSpec((kblk, H), lambda b, k: (k, 0)),
            pl.BlockSpec((1, H), lambda b, k: (0, 0)),
            pl.BlockSpec((1, H), lambda b, k: (0, 0)),
            pl.BlockSpec((1, H), lambda b, k: (0, 0)),
        ],
        out_specs=pl.BlockSpec((bblk, H), lambda b, k: (b, 0)),
        out_shape=jax.ShapeDtypeStruct((B, H), jnp.float32),
        scratch_shapes=[pltpu.VMEM((bblk, H), jnp.float32)],
    )(x, w, b2d, g2d, beta2d)


def kernel(query_embed, kb, Wq, bq, W1, b1, g1, beta1, W2, b2, g2, beta2):
    B, D = query_embed.shape
    H = W1.shape[1]
    qn = _qproj(query_embed, Wq, bq.reshape(1, D))
    vals, idx = _sim_topk(qn, kb)
    rows = _sc_gather(kb, idx.reshape(B * TK))
    flat = rows.reshape(B, TK * D)
    h1 = _mlp(flat, W1, b1.reshape(1, H), g1.reshape(1, H),
              beta1.reshape(1, H), gelu=True)
    ctx = _mlp(h1, W2, b2.reshape(1, H), g2.reshape(1, H),
               beta2.reshape(1, H), gelu=False)
    return (ctx, idx, vals)


# speedup vs baseline: 2.4749x; 1.2023x over previous
import functools

import jax
import jax.numpy as jnp
from jax import lax
from jax.experimental import pallas as pl
from jax.experimental.pallas import tpu as pltpu
from jax.experimental.pallas import tpu_sc as plsc

TK = 8
_KBLK = 2048
_NGRP = 4


def _qproj_body(x_ref, w_ref, b_ref, o_ref):
    q = jnp.dot(x_ref[...], w_ref[...], preferred_element_type=jnp.float32)
    q = q + b_ref[...]
    n = jnp.sqrt(jnp.sum(q * q, axis=1, keepdims=True))
    o_ref[...] = q / n


def _qproj(x, w, b2d):
    B, D = x.shape
    return pl.pallas_call(
        _qproj_body,
        out_shape=jax.ShapeDtypeStruct((B, D), jnp.float32),
    )(x, w, b2d)


def _sim_topk_body(qn_ref, kb_ref, vals_ref, idx_ref, xs, cand_v, cand_i,
                   *, B, K, kblk):
    k = pl.program_id(0)

    @pl.when(k == 0)
    def _init():
        vals_ref[...] = jnp.full((B, TK), -jnp.inf, jnp.float32)
        idx_ref[...] = jnp.zeros((B, TK), jnp.int32)

    kbb = kb_ref[...]
    n2 = jnp.sum(kbb * kbb, axis=1, keepdims=True)
    kbn = kbb / jnp.sqrt(n2)
    s = lax.dot_general(qn_ref[...], kbn, (((1,), (1,)), ((), ())),
                        preferred_element_type=jnp.float32)

    lane_f = lax.broadcasted_iota(jnp.int32, (1, kblk), 1).astype(jnp.float32)
    col_ok = lane_f < (K - k * kblk).astype(jnp.float32)
    xs[...] = jnp.where(col_ok, s, -jnp.inf)

    g = B // _NGRP
    lane8f = lax.broadcasted_iota(jnp.int32, (1, TK), 1).astype(jnp.float32)
    Ms = []
    for gi in range(_NGRP):
        r0, r1 = gi * g, (gi + 1) * g
        rmin = vals_ref[r0:r1, TK - 1:TK]
        cnt = jnp.sum((xs[r0:r1, :] > rmin).astype(jnp.float32), axis=1)
        Mg = jnp.max(cnt)
        Ms.append(Mg)
        cand_v[r0:r1, :] = jnp.full((g, TK), -jnp.inf, jnp.float32)
        cand_i[r0:r1, :] = jnp.zeros((g, TK), jnp.float32)
        for t in range(TK):
            @pl.when(t < Mg)
            def _round(r0=r0, r1=r1, t=t):
                x = xs[r0:r1, :]
                m = jnp.max(x, axis=1, keepdims=True)
                am = jnp.min(jnp.where(x == m, lane_f, float(kblk)),
                             axis=1, keepdims=True)
                cand_v[r0:r1, :] = jnp.where(lane8f == t, m, cand_v[r0:r1, :])
                cand_i[r0:r1, :] = jnp.where(lane8f == t, am, cand_i[r0:r1, :])
                xs[r0:r1, :] = jnp.where(lane_f == am, -jnp.inf, x)

    Mall = Ms[0]
    for Mg in Ms[1:]:
        Mall = jnp.maximum(Mall, Mg)

    @pl.when(Mall > 0)
    def _merge():
        cv = jnp.concatenate([vals_ref[...], cand_v[...]], axis=1)
        ci = jnp.concatenate([idx_ref[...].astype(jnp.float32),
                              cand_i[...] + jnp.float32(k * kblk)], axis=1)
        lane16 = lax.broadcasted_iota(jnp.int32,
                                      (1, 2 * TK), 1).astype(jnp.float32)
        nv, ni = [], []
        for _ in range(TK):
            m = jnp.max(cv, axis=1, keepdims=True)
            am = jnp.min(jnp.where(cv == m, lane16, float(2 * TK)), axis=1,
                         keepdims=True)
            pick = lane16 == am
            nv.append(m)
            ni.append(jnp.max(jnp.where(pick, ci, -1.0), axis=1,
                              keepdims=True))
            cv = jnp.where(pick, -jnp.inf, cv)
        vals_ref[...] = jnp.concatenate(nv, axis=1)
        idx_ref[...] = jnp.concatenate(ni, axis=1).astype(jnp.int32)


def _sim_topk(qn, kb):
    B, D = qn.shape
    K = kb.shape[0]
    nk = pl.cdiv(K, _KBLK)
    body = functools.partial(_sim_topk_body, B=B, K=K, kblk=_KBLK)
    return pl.pallas_call(
        body,
        grid=(nk,),
        in_specs=[
            pl.BlockSpec((B, D), lambda k: (0, 0)),
            pl.BlockSpec((_KBLK, D), lambda k: (k, 0)),
        ],
        out_specs=[
            pl.BlockSpec((B, TK), lambda k: (0, 0)),
            pl.BlockSpec((B, TK), lambda k: (0, 0)),
        ],
        out_shape=[
            jax.ShapeDtypeStruct((B, TK), jnp.float32),
            jax.ShapeDtypeStruct((B, TK), jnp.int32),
        ],
        scratch_shapes=[
            pltpu.VMEM((B, _KBLK), jnp.float32),
            pltpu.VMEM((B, TK), jnp.float32),
            pltpu.VMEM((B, TK), jnp.float32),
        ],
    )(qn, kb)


def _sc_gather(table, idx_flat):
    V, D = table.shape
    N = idx_flat.shape[0]
    info = plsc.get_sparse_core_info()
    nw = info.num_cores * info.num_subcores
    b_per_w = N // nw
    chunk = 128
    mesh = plsc.VectorSubcoreMesh(core_axis_name="c", subcore_axis_name="s")

    @functools.partial(
        pl.kernel,
        mesh=mesh,
        out_type=jax.ShapeDtypeStruct((N, D), jnp.float32),
        scratch_types=[
            pltpu.VMEM((chunk,), jnp.int32),
            pltpu.VMEM((chunk, D), jnp.float32),
            pltpu.SemaphoreType.DMA,
        ],
    )
    def k(table_hbm, idx_hbm, out_hbm, idx_v, rows_v, sem):
        wid = lax.axis_index("s") * info.num_cores + lax.axis_index("c")
        base = wid * b_per_w
        for c in range(b_per_w // chunk):
            off = base + c * chunk
            pltpu.sync_copy(idx_hbm.at[pl.ds(off, chunk)], idx_v)
            pltpu.async_copy(table_hbm.at[idx_v], rows_v, sem).wait()
            pltpu.sync_copy(rows_v, out_hbm.at[pl.ds(off, chunk)])

    return k(table, idx_flat)


def _mlp_body(x_ref, w_ref, b_ref, g_ref, beta_ref, o_ref, acc, *, gelu):
    kk = pl.program_id(1)
    nk = pl.num_programs(1)

    @pl.when(kk == 0)
    def _init():
        acc[...] = jnp.zeros_like(acc)

    acc[...] += jnp.dot(x_ref[...].astype(jnp.bfloat16),
                        w_ref[...].astype(jnp.bfloat16),
                        preferred_element_type=jnp.float32)

    @pl.when(kk == nk - 1)
    def _fin():
        h = acc[...] + b_ref[...]
        mu = jnp.mean(h, axis=1, keepdims=True)
        var = jnp.mean((h - mu) ** 2, axis=1, keepdims=True)
        h = (h - mu) / jnp.sqrt(var + 1e-5) * g_ref[...] + beta_ref[...]
        if gelu:
            h = 0.5 * h * (1.0 + lax.erf(h / jnp.sqrt(2.0).astype(jnp.float32)))
        o_ref[...] = h


def _mlp(x, w, b2d, g2d, beta2d, gelu):
    B, Kin = x.shape
    H = w.shape[1]
    bblk, kblk = min(256, B), min(1024, Kin)
    grid = (B // bblk, Kin // kblk)
    body = functools.partial(_mlp_body, gelu=gelu)
    return pl.pallas_call(
        body,
        grid=grid,
        in_specs=[
            pl.BlockSpec((bblk, kblk), lambda b, k: (b, k)),
            pl.BlockSpec((kblk, H), lambda b, k: (k, 0)),
            pl.BlockSpec((1, H), lambda b, k: (0, 0)),
            pl.BlockSpec((1, H), lambda b, k: (0, 0)),
            pl.BlockSpec((1, H), lambda b, k: (0, 0)),
        ],
        out_specs=pl.BlockSpec((bblk, H), lambda b, k: (b, 0)),
        out_shape=jax.ShapeDtypeStruct((B, H), jnp.float32),
        scratch_shapes=[pltpu.VMEM((bblk, H), jnp.float32)],
    )(x, w, b2d, g2d, beta2d)


def kernel(query_embed, kb, Wq, bq, W1, b1, g1, beta1, W2, b2, g2, beta2):
    B, D = query_embed.shape
    H = W1.shape[1]
    qn = _qproj(query_embed, Wq, bq.reshape(1, D))
    vals, idx = _sim_topk(qn, kb)
    rows = _sc_gather(kb, idx.reshape(B * TK))
    flat = rows.reshape(B, TK * D)
    h1 = _mlp(flat, W1, b1.reshape(1, H), g1.reshape(1, H),
              beta1.reshape(1, H), gelu=True)
    ctx = _mlp(h1, W2, b2.reshape(1, H), g2.reshape(1, H),
               beta2.reshape(1, H), gelu=False)
    return (ctx, idx, vals)
